# Initial kernel scaffold; baseline (speedup 1.0000x reference)
import jax
import jax.numpy as jnp
from jax.experimental import pallas as pl


def kernel(cur, ctx, neg, W_in, W_out):
    raise NotImplementedError("write your pallas kernel here")


# speedup vs baseline: 1.6540x; 1.6540x over previous
import functools

import jax
import jax.numpy as jnp
from jax import lax
from jax.experimental import pallas as pl
from jax.experimental.pallas import tpu as pltpu
from jax.experimental.pallas import tpu_sc as plsc

B, NCTX, NNEGS, V, D = 4096, 20, 5, 1000000, 64
NSAM = (1 + NNEGS) * NCTX
NPAD = 128
NG = NPAD // 16
NW = 32
BPW = B // NW
L = 16


def _sc_sim_body(cur_h, ctx_h, neg_h, win_h, wout_h, out_h,
                 curi_v, ctxi_v, negi_v, currow_v, rows_v, sim_v, sem):
    wid = lax.axis_index("s") * 2 + lax.axis_index("c")
    base = wid * BPW

    pltpu.sync_copy(cur_h.at[pl.ds(base, BPW)], curi_v)
    pltpu.sync_copy(ctx_h.at[pl.ds(base, BPW)], ctxi_v)
    pltpu.sync_copy(neg_h.at[pl.ds(base, BPW)], negi_v)
    pltpu.async_copy(win_h.at[curi_v], currow_v, sem).wait()

    zero = jnp.zeros((L,), jnp.float32)
    for r in range(NSAM, NPAD):
        for q in range(D // L):
            rows_v[r, pl.ds(q * L, L)] = zero

    lanes = lax.iota(jnp.int32, L)
    samp = [lanes + g * L for g in range(NG)]
    sign = [jnp.where(lanes + g * L < NCTX, 1.0, -1.0).astype(jnp.float32)
            for g in range(NG)]
    def body(b, carry):
        c1 = pltpu.async_copy(wout_h.at[ctxi_v.at[b]],
                              rows_v.at[pl.ds(0, NCTX)], sem)
        c2 = pltpu.async_copy(wout_h.at[negi_v.at[b]],
                              rows_v.at[pl.ds(NCTX, NSAM - NCTX)], sem)
        c1.wait()
        c2.wait()

        accs = [zero] * NG
        for q in range(D // L):
            cvec = currow_v[b, pl.ds(q * L, L)]
            for dd in range(L):
                s = cvec[dd]
                col = jnp.full((L,), q * L + dd, jnp.int32)
                for g in range(NG):
                    accs[g] = accs[g] + plsc.load_gather(
                        rows_v, [samp[g], col]) * s
        for g in range(NG):
            sim_v[b, pl.ds(g * L, L)] = accs[g] * sign[g]
        return carry

    lax.fori_loop(0, BPW, body, 0)
    pltpu.sync_copy(sim_v, out_h.at[pl.ds(base, BPW)])


def _tc_loss_body(sim_ref, out_ref):
    x = sim_ref[...]
    col = lax.broadcasted_iota(jnp.int32, x.shape, 1)
    ls = jax.nn.log_sigmoid(x)
    out_ref[0, 0] = -jnp.sum(jnp.where(col < NSAM, ls, 0.0)) / B


def kernel(cur, ctx, neg, W_in, W_out):
    cur = cur.astype(jnp.int32)
    ctx = ctx.astype(jnp.int32)
    neg = neg.astype(jnp.int32)

    sc_sim = functools.partial(
        pl.kernel,
        out_type=jax.ShapeDtypeStruct((B, NPAD), jnp.float32),
        mesh=plsc.VectorSubcoreMesh(core_axis_name="c", subcore_axis_name="s"),
        scratch_types=[
            pltpu.VMEM((BPW,), jnp.int32),
            pltpu.VMEM((BPW, NCTX), jnp.int32),
            pltpu.VMEM((BPW, NSAM - NCTX), jnp.int32),
            pltpu.VMEM((BPW, D), jnp.float32),
            pltpu.VMEM((NPAD, D), jnp.float32),
            pltpu.VMEM((BPW, NPAD), jnp.float32),
            pltpu.SemaphoreType.DMA,
        ],
        compiler_params=pltpu.CompilerParams(
            needs_layout_passes=False, use_tc_tiling_on_sc=False),
    )(_sc_sim_body)

    sim = sc_sim(cur, ctx, neg, W_in, W_out)

    loss = pl.pallas_call(
        _tc_loss_body,
        out_shape=jax.ShapeDtypeStruct((1, 1), jnp.float32),
        out_specs=pl.BlockSpec(memory_space=pltpu.SMEM),
    )(sim)
    return loss[0, 0]
